# baseline (speedup 1.0000x reference)
import jax
import jax.numpy as jnp
from jax.experimental import pallas as pl
from jax.experimental.pallas import tpu as pltpu
from jax.sharding import PartitionSpec as P


_BM = 5000


def _mm_body(q_ref, mk_ref, o_ref):
    mk = mk_ref[...].astype(jnp.bfloat16)
    qb = q_ref[...].astype(jnp.bfloat16)
    o_ref[...] = jnp.dot(mk, qb, preferred_element_type=jnp.float32)


def _mm(q, memory_key):
    m, k = memory_key.shape
    b = q.shape[1]
    bm = _BM if m % _BM == 0 else m
    return pl.pallas_call(
        _mm_body,
        grid=(m // bm,),
        in_specs=[
            pl.BlockSpec((k, b), lambda i: (0, 0)),
            pl.BlockSpec((bm, k), lambda i: (i, 0)),
        ],
        out_specs=pl.BlockSpec((bm, b), lambda i: (i, 0)),
        out_shape=jax.ShapeDtypeStruct((m, b), jnp.float32),
        compiler_params=pltpu.CompilerParams(
            dimension_semantics=("parallel",),
        ),
    )(q, memory_key)


def kernel(q, memory_key):
    m = memory_key.shape[0]
    n_dev = len(jax.devices())
    p = n_dev
    while p > 1 and (m % (p * _BM) != 0):
        p -= 1
    if p <= 1:
        return _mm(q, memory_key)
    mesh = jax.make_mesh((p,), ("x",))
    q_sh = jax.reshard(q, jax.sharding.NamedSharding(mesh, P(None, None)))
    mk_sh = jax.reshard(memory_key, jax.sharding.NamedSharding(mesh, P("x", None)))
    fn = jax.shard_map(
        _mm,
        mesh=mesh,
        in_specs=(P(None, None), P("x", None)),
        out_specs=P("x", None),
        check_vma=False,
    )
    return fn(q_sh, mk_sh)


# speedup vs baseline: 2.6667x; 2.6667x over previous
import jax
import jax.numpy as jnp
from jax.experimental import pallas as pl
from jax.experimental.pallas import tpu as pltpu


_BM = 5000


def _mm_body(q_ref, mk_ref, o_ref):
    mk = mk_ref[...].astype(jnp.bfloat16)
    qb = q_ref[...].astype(jnp.bfloat16)
    o_ref[...] = jnp.dot(mk, qb, preferred_element_type=jnp.float32)


def kernel(q, memory_key):
    m, k = memory_key.shape
    b = q.shape[1]
    bm = _BM if m % _BM == 0 else m
    return pl.pallas_call(
        _mm_body,
        grid=(m // bm,),
        in_specs=[
            pl.BlockSpec((k, b), lambda i: (0, 0)),
            pl.BlockSpec((bm, k), lambda i: (i, 0)),
        ],
        out_specs=pl.BlockSpec((bm, b), lambda i: (i, 0)),
        out_shape=jax.ShapeDtypeStruct((m, b), jnp.float32),
        compiler_params=pltpu.CompilerParams(
            dimension_semantics=("parallel",),
        ),
    )(q, memory_key)
